# Initial kernel scaffold; baseline (speedup 1.0000x reference)
import jax
import jax.numpy as jnp
from jax.experimental import pallas as pl


def kernel(rtdetr_raw_out):
    raise NotImplementedError("write your pallas kernel here")


# speedup vs baseline: 27.2377x; 27.2377x over previous
import functools

import jax
import jax.numpy as jnp
from jax import lax
from jax.experimental import pallas as pl
from jax.experimental.pallas import tpu as pltpu

_C = 128


def _nms_body(n, cx_ref, cy_ref, w_ref, h_ref, keep_ref,
              x1_s, y1_s, x2_s, y2_s, area_s, gidx_s):
    rows = cx_ref.shape[0]
    cxs = cx_ref[:] * 640.0
    cys = cy_ref[:] * 640.0
    ws = w_ref[:] * 640.0
    hs = h_ref[:] * 640.0
    x1 = cxs - ws / 2.0
    y1 = cys - hs / 2.0
    x2 = cxs + ws / 2.0
    y2 = cys + hs / 2.0
    x1_s[:] = x1
    y1_s[:] = y1
    x2_s[:] = x2
    y2_s[:] = y2
    area_s[:] = (x2 - x1) * (y2 - y1)
    ri = lax.broadcasted_iota(jnp.int32, (rows, _C), 0)
    ci = lax.broadcasted_iota(jnp.int32, (rows, _C), 1)
    gidx_s[:] = ri * _C + ci
    keep_ref[:] = jnp.zeros((rows, _C), jnp.float32)

    lane = lax.broadcasted_iota(jnp.int32, (1, _C), 1)

    def body(i, carry):
        r = i // _C
        l = i % _C
        onehot = lane == l
        sup_row = keep_ref[pl.ds(r, 1), :]
        sup_i = jnp.max(jnp.where(onehot, sup_row, 0.0))

        @pl.when(sup_i == 0.0)
        def _():
            bx1 = jnp.max(jnp.where(onehot, x1_s[pl.ds(r, 1), :], -3.0e38))
            by1 = jnp.max(jnp.where(onehot, y1_s[pl.ds(r, 1), :], -3.0e38))
            bx2 = jnp.max(jnp.where(onehot, x2_s[pl.ds(r, 1), :], -3.0e38))
            by2 = jnp.max(jnp.where(onehot, y2_s[pl.ds(r, 1), :], -3.0e38))
            bar = jnp.max(jnp.where(onehot, area_s[pl.ds(r, 1), :], -3.0e38))
            xx1 = jnp.maximum(x1_s[:], bx1)
            yy1 = jnp.maximum(y1_s[:], by1)
            xx2 = jnp.minimum(x2_s[:], bx2)
            yy2 = jnp.minimum(y2_s[:], by2)
            inter = jnp.maximum(xx2 - xx1, 0.0) * jnp.maximum(yy2 - yy1, 0.0)
            union = bar + area_s[:] - inter
            iou = inter / union
            supnew = (iou > 0.5) & (gidx_s[:] > i)
            keep_ref[:] = jnp.where(supnew, 1.0, keep_ref[:])

        return carry

    lax.fori_loop(0, n, body, 0)
    keep_ref[:] = 1.0 - keep_ref[:]


def kernel(rtdetr_raw_out):
    x = jnp.squeeze(rtdetr_raw_out, axis=0)
    n = x.shape[0]
    npad = ((n + 1023) // 1024) * 1024
    rows = npad // _C

    conf = x[:, 4]
    confn = conf / jnp.max(conf)
    order = jnp.argsort(-confn)
    xs = x[order]
    xs = jnp.concatenate([xs, jnp.zeros((npad - n, 5), jnp.float32)], axis=0)
    cx = xs[:, 0].reshape(rows, _C)
    cy = xs[:, 1].reshape(rows, _C)
    w = xs[:, 2].reshape(rows, _C)
    h = xs[:, 3].reshape(rows, _C)

    keep_f = pl.pallas_call(
        functools.partial(_nms_body, n),
        out_shape=jax.ShapeDtypeStruct((rows, _C), jnp.float32),
        scratch_shapes=[pltpu.VMEM((rows, _C), jnp.float32)] * 5
        + [pltpu.VMEM((rows, _C), jnp.int32)],
    )(cx, cy, w, h)

    keep_sorted = keep_f.reshape(-1)[:n] > 0.5
    keep = jnp.zeros((n,), bool).at[order].set(keep_sorted)

    cxcywh = x[:, :4] * 640.0
    ucx, ucy, uw, uh = cxcywh[:, 0], cxcywh[:, 1], cxcywh[:, 2], cxcywh[:, 3]
    xyxy = jnp.stack(
        [ucx - uw / 2.0, ucy - uh / 2.0, ucx + uw / 2.0, ucy + uh / 2.0], axis=-1
    )
    boxes_and_scores = jnp.concatenate([xyxy, confn[:, None]], axis=1)
    mask = keep & (boxes_and_scores[:, 4] >= 0.25)
    return jnp.where(mask[:, None], boxes_and_scores, 0.0)


# speedup vs baseline: 32.0489x; 1.1766x over previous
import functools

import jax
import jax.numpy as jnp
from jax import lax
from jax.experimental import pallas as pl
from jax.experimental.pallas import tpu as pltpu
from jax.experimental.pallas import tpu_sc as plsc

_N = 20000
_NPAD = 20480
_C = 128
_ROWS = _NPAD // _C
_NW = 32
_BPW = _NPAD // _NW
_IR = _BPW // _C
_D = 16

def _worker_id():
    return lax.axis_index("s") * 2 + lax.axis_index("c")


@functools.lru_cache(maxsize=1)
def _build_sc_kernels():
    mesh = plsc.VectorSubcoreMesh(
        core_axis_name="c", subcore_axis_name="s", num_cores=2, num_subcores=16
    )
    out16 = jax.ShapeDtypeStruct((_NPAD, _D), jnp.float32)
    scratch = [
        pltpu.VMEM((_IR, _C), jnp.int32),
        pltpu.VMEM((_BPW, _D), jnp.float32),
        pltpu.SemaphoreType.DMA,
    ]

    @functools.partial(
        pl.kernel, out_type=out16, mesh=mesh, scratch_types=scratch,
        compiler_params=pltpu.CompilerParams(use_tc_tiling_on_sc=False),
    )
    def gather(table_hbm, idx_hbm, out_hbm, idx_v, rows_v, sem):
        wid = _worker_id()
        pltpu.sync_copy(idx_hbm.at[wid], idx_v)
        cps = [
            pltpu.async_copy(
                table_hbm.at[idx_v.at[c]], rows_v.at[pl.ds(c * _C, _C)], sem
            )
            for c in range(_IR)
        ]
        for cp in cps:
            cp.wait()
        pltpu.sync_copy(rows_v, out_hbm.at[pl.ds(wid * _BPW, _BPW)])

    @functools.partial(
        pl.kernel, out_type=out16, mesh=mesh, scratch_types=scratch,
        compiler_params=pltpu.CompilerParams(use_tc_tiling_on_sc=False),
    )
    def scatter(rows_hbm, idx_hbm, out_hbm, idx_v, rows_v, sem):
        wid = _worker_id()
        pltpu.sync_copy(idx_hbm.at[wid], idx_v)
        pltpu.sync_copy(rows_hbm.at[pl.ds(wid * _BPW, _BPW)], rows_v)
        cps = [
            pltpu.async_copy(
                rows_v.at[pl.ds(c * _C, _C)], out_hbm.at[idx_v.at[c]], sem
            )
            for c in range(_IR)
        ]
        for cp in cps:
            cp.wait()

    return gather, scatter


def _sc_gather(table, idx):
    return _build_sc_kernels()[0](table, idx)


def _sc_scatter(rows16, idx):
    return _build_sc_kernels()[1](rows16, idx)


def _nms_body(cx_ref, cy_ref, w_ref, h_ref, cf_ref,
              keep_ref, ox1, oy1, ox2, oy2, ocf,
              x1_s, y1_s, x2_s, y2_s, area_s, gidx_s):
    cxs = cx_ref[:] * 640.0
    cys = cy_ref[:] * 640.0
    ws = w_ref[:] * 640.0
    hs = h_ref[:] * 640.0
    x1 = cxs - ws / 2.0
    y1 = cys - hs / 2.0
    x2 = cxs + ws / 2.0
    y2 = cys + hs / 2.0
    x1_s[:] = x1
    y1_s[:] = y1
    x2_s[:] = x2
    y2_s[:] = y2
    area_s[:] = (x2 - x1) * (y2 - y1)
    ri = lax.broadcasted_iota(jnp.int32, (_ROWS, _C), 0)
    ci = lax.broadcasted_iota(jnp.int32, (_ROWS, _C), 1)
    gidx_s[:] = ri * _C + ci
    keep_ref[:] = jnp.zeros((_ROWS, _C), jnp.float32)

    cf = cf_ref[:]
    thr_mask = cf >= 0.25
    m = jnp.sum(thr_mask.astype(jnp.int32))
    cmax = (m + 1023) // 1024

    lane = lax.broadcasted_iota(jnp.int32, (1, _C), 1)

    def body(i, carry):
        r = i // _C
        l = i % _C
        onehot = lane == l
        sup_row = keep_ref[pl.ds(r, 1), :]
        sup_i = jnp.max(jnp.where(onehot, sup_row, 0.0))

        @pl.when(sup_i == 0.0)
        def _():
            bx1 = jnp.max(jnp.where(onehot, x1_s[pl.ds(r, 1), :], -3.0e38))
            by1 = jnp.max(jnp.where(onehot, y1_s[pl.ds(r, 1), :], -3.0e38))
            bx2 = jnp.max(jnp.where(onehot, x2_s[pl.ds(r, 1), :], -3.0e38))
            by2 = jnp.max(jnp.where(onehot, y2_s[pl.ds(r, 1), :], -3.0e38))
            bar = jnp.max(jnp.where(onehot, area_s[pl.ds(r, 1), :], -3.0e38))

            def chunk(c, carry2):
                sl = pl.ds(c * 8, 8)
                xx1 = jnp.maximum(x1_s[sl, :], bx1)
                yy1 = jnp.maximum(y1_s[sl, :], by1)
                xx2 = jnp.minimum(x2_s[sl, :], bx2)
                yy2 = jnp.minimum(y2_s[sl, :], by2)
                inter = (jnp.maximum(xx2 - xx1, 0.0)
                         * jnp.maximum(yy2 - yy1, 0.0))
                union = bar + area_s[sl, :] - inter
                iou = inter / union
                supnew = (iou > 0.5) & (gidx_s[sl, :] > i)
                keep_ref[sl, :] = jnp.where(supnew, 1.0, keep_ref[sl, :])
                return carry2

            lax.fori_loop(r // 8, cmax, chunk, 0)

        return carry

    lax.fori_loop(0, m, body, 0)

    keepv = keep_ref[:] == 0.0
    keep_ref[:] = keepv.astype(jnp.float32)
    outm = keepv & thr_mask
    ox1[:] = jnp.where(outm, x1_s[:], 0.0)
    oy1[:] = jnp.where(outm, y1_s[:], 0.0)
    ox2[:] = jnp.where(outm, x2_s[:], 0.0)
    oy2[:] = jnp.where(outm, y2_s[:], 0.0)
    ocf[:] = jnp.where(outm, cf, 0.0)


_plane = jax.ShapeDtypeStruct((_ROWS, _C), jnp.float32)

_nms_call = pl.pallas_call(
    _nms_body,
    out_shape=[_plane] * 6,
    scratch_shapes=[pltpu.VMEM((_ROWS, _C), jnp.float32)] * 5
    + [pltpu.VMEM((_ROWS, _C), jnp.int32)],
)


def kernel(rtdetr_raw_out):
    x = jnp.squeeze(rtdetr_raw_out, axis=0)
    conf = x[:, 4]
    confn = conf / jnp.max(conf)
    order = jnp.argsort(-confn)

    table = jnp.concatenate(
        [x[:, :4], confn[:, None], jnp.zeros((_N, _D - 5), jnp.float32)], axis=1
    )
    table = jnp.concatenate([table, jnp.zeros((_NPAD - _N, _D), jnp.float32)])
    idx = jnp.concatenate(
        [order.astype(jnp.int32), jnp.arange(_N, _NPAD, dtype=jnp.int32)]
    ).reshape(_NW, _IR, _C)

    sorted16 = _sc_gather(table, idx)

    planes = [sorted16[:, c].reshape(_ROWS, _C) for c in range(5)]
    _keep, px1, py1, px2, py2, pcf = _nms_call(*planes)

    rows5 = jnp.stack(
        [p.reshape(-1) for p in (px1, py1, px2, py2, pcf)], axis=-1
    )
    rows16 = jnp.concatenate(
        [rows5, jnp.zeros((_NPAD, _D - 5), jnp.float32)], axis=1
    )
    out16 = _sc_scatter(rows16, idx)
    return out16[:_N, :5]
